# baseline (speedup 1.0000x reference)
import functools
import math

import jax
import jax.numpy as jnp
import numpy as np
from jax import lax
from jax.experimental import pallas as pl
from jax.experimental.pallas import tpu as pltpu
from jax.experimental.pallas import tpu_sc as plsc

VOCAB = 100000
D = 128
L_TOK = 24
B = 4096
MAX_LEN = 25


def _pe_const(max_len, d_model):
    position = np.arange(0, max_len, dtype=np.float32)[:, None]
    div_term = np.exp(
        np.arange(0, d_model, 2).astype(np.float32) * (-math.log(10000.0) / d_model)
    )
    pe = np.zeros((max_len, d_model), dtype=np.float32)
    pe[:, 0::2] = np.sin(position * div_term)
    pe[:, 1::2] = np.cos(position * div_term)
    return pe


_PE = _pe_const(MAX_LEN, D)

_INFO = plsc.get_sparse_core_info()
_NC = _INFO.num_cores
_NS = _INFO.num_subcores
_NW = _NC * _NS

_B_PER_W = B // _NW
_NB = 8
_CHUNKS = _B_PER_W // _NB
_TOK = _NB * L_TOK
_WTOK = _B_PER_W * L_TOK
_HALF = _TOK // 2
_OROWS = _NB * MAX_LEN
_GRPS = _TOK // 16

_PROW = np.array([t + t // L_TOK + 1 for t in range(_TOK)], dtype=np.int32)


def _sc_body(words_hbm, sidx_hbm, noun_hbm, addt_hbm, spec_hbm, prow_hbm,
             out_hbm,
             widx_v, sidx_v, addt_v, prow_v, spec_v,
             g0, g1, o0, o1,
             sem_g0, sem_g1, sem_w0, sem_w1):
    wid = lax.axis_index("s") * _NC + lax.axis_index("c")
    wtok = wid * _WTOK

    def fire_gathers(k, g, sem):
        cps = []
        for h in range(2):
            cps.append(pltpu.async_copy(
                noun_hbm.at[widx_v.at[pl.ds(k * _TOK + h * _HALF, _HALF)]],
                g.at[pl.ds(h * _HALF, _HALF)], sem))
        return cps

    def drain_gathers(g, sem):
        for h in range(2):
            pltpu.make_async_copy(
                noun_hbm.at[pl.ds(0, _HALF)],
                g.at[pl.ds(h * _HALF, _HALF)], sem).wait()

    def fire_writes(k, o, sem):
        row0 = (wid * _B_PER_W + k * _NB) * MAX_LEN
        pltpu.async_copy(o, out_hbm.at[pl.ds(row0, _OROWS)], sem)

    def drain_writes(o, sem):
        pltpu.make_async_copy(o, out_hbm.at[pl.ds(0, _OROWS)], sem).wait()

    def add_assemble(k, g, o):
        def grp(gi, carry):
            s16 = sidx_v[pl.ds(k * _TOK + gi * 16, 16)]
            r16 = prow_v[pl.ds(gi * 16, 16)]
            t16 = lax.broadcasted_iota(jnp.int32, (16,), 0) + gi * 16

            def colblk(cb, carry2):
                for q in range(16):
                    cs = jnp.zeros((16,), jnp.int32) + (cb * 16 + q)
                    av = plsc.load_gather(addt_v, [s16, cs])
                    gv = plsc.load_gather(g, [t16, cs])
                    plsc.store_scatter(o, [r16, cs], av + gv)
                return carry2

            lax.fori_loop(0, D // 16, colblk, 0)
            return carry

        lax.fori_loop(0, _GRPS, grp, 0)

    def process(k, g, o, semg, semw, gn, semgn, fire_next, drain_w):
        pl.when(fire_next)(lambda: (fire_gathers(k + 1, gn, semgn), None)[1])
        drain_gathers(g, semg)
        pl.when(drain_w)(lambda: drain_writes(o, semw))
        add_assemble(k, g, o)
        fire_writes(k, o, semw)

    pltpu.sync_copy(words_hbm.at[pl.ds(wtok, _WTOK)], widx_v)
    pltpu.sync_copy(sidx_hbm.at[pl.ds(wtok, _WTOK)], sidx_v)
    pltpu.sync_copy(addt_hbm, addt_v)
    pltpu.sync_copy(prow_hbm, prow_v)
    pltpu.sync_copy(spec_hbm, spec_v)
    for ov in (o0, o1):
        for j in range(_NB):
            for q in range(D // 16):
                sl = pl.ds(q * 16, 16)
                ov[j * MAX_LEN, sl] = spec_v[0, sl]
    fire_gathers(0, g0, sem_g0)

    true_ = jnp.bool_(True)

    def pair(i, carry):
        k = 2 * i
        late = i >= 1
        process(k, g0, o0, sem_g0, sem_w0, g1, sem_g1, true_, late)
        process(k + 1, g1, o1, sem_g1, sem_w1, g0, sem_g0, i < _CHUNKS // 2 - 1,
                late)
        return carry

    lax.fori_loop(0, _CHUNKS // 2, pair, 0)
    drain_writes(o0, sem_w0)
    drain_writes(o1, sem_w1)


def kernel(words, classes, noun_table, class_table, special_emb):
    words_flat = words.astype(jnp.int32).reshape(-1)
    sidx = (2 * jnp.arange(L_TOK, dtype=jnp.int32)[None, :]
            + classes.astype(jnp.int32)).reshape(-1)
    pe = jnp.asarray(_PE[:L_TOK])
    addt = (pe[:, None, :] + class_table[None, :, :]).reshape(2 * L_TOK, D)
    prow = jnp.asarray(_PROW)

    mesh = plsc.VectorSubcoreMesh(core_axis_name="c", subcore_axis_name="s")
    run = functools.partial(
        pl.kernel,
        mesh=mesh,
        compiler_params=pltpu.CompilerParams(
            use_tc_tiling_on_sc=False, needs_layout_passes=False),
        out_type=jax.ShapeDtypeStruct((B * MAX_LEN, D), jnp.float32),
        scratch_types=[
            pltpu.VMEM((_WTOK,), jnp.int32),
            pltpu.VMEM((_WTOK,), jnp.int32),
            pltpu.VMEM((2 * L_TOK, D), jnp.float32),
            pltpu.VMEM((_TOK,), jnp.int32),
            pltpu.VMEM((1, D), jnp.float32),
            pltpu.VMEM((_TOK, D), jnp.float32),
            pltpu.VMEM((_TOK, D), jnp.float32),
            pltpu.VMEM((_OROWS, D), jnp.float32),
            pltpu.VMEM((_OROWS, D), jnp.float32),
            pltpu.SemaphoreType.DMA,
            pltpu.SemaphoreType.DMA,
            pltpu.SemaphoreType.DMA,
            pltpu.SemaphoreType.DMA,
        ],
    )(_sc_body)
    out = run(words_flat, sidx, noun_table, addt, special_emb, prow)
    return out.reshape(B, MAX_LEN, D)


# speedup vs baseline: 2.9196x; 2.9196x over previous
import functools
import math

import jax
import jax.numpy as jnp
import numpy as np
from jax import lax
from jax.experimental import pallas as pl
from jax.experimental.pallas import tpu as pltpu
from jax.experimental.pallas import tpu_sc as plsc

VOCAB = 100000
D = 128
L_TOK = 24
B = 4096
MAX_LEN = 25


def _pe_const(max_len, d_model):
    position = np.arange(0, max_len, dtype=np.float32)[:, None]
    div_term = np.exp(
        np.arange(0, d_model, 2).astype(np.float32) * (-math.log(10000.0) / d_model)
    )
    pe = np.zeros((max_len, d_model), dtype=np.float32)
    pe[:, 0::2] = np.sin(position * div_term)
    pe[:, 1::2] = np.cos(position * div_term)
    return pe


_PE = _pe_const(MAX_LEN, D)

_INFO = plsc.get_sparse_core_info()
_NC = _INFO.num_cores
_NS = _INFO.num_subcores
_NW = _NC * _NS

_B_PER_W = B // _NW
_NB = 8
_CHUNKS = _B_PER_W // _NB
_TOK = _NB * L_TOK
_WTOK = _B_PER_W * L_TOK
_HALF = _TOK // 2
_OROWS = _NB * MAX_LEN

_ONEHOT = [np.eye(16, dtype=np.bool_)[i] for i in range(16)]


def _sc_body(words_hbm, cls_hbm, noun_hbm, base_hbm, dvec_hbm, spec_hbm,
             out_hbm,
             widx_v, cls_v, base_v, dvec_v, spec_v,
             g0, g1, o0, o1,
             sem_g0, sem_g1, sem_w0, sem_w1):
    wid = lax.axis_index("s") * _NC + lax.axis_index("c")
    wtok = wid * _WTOK

    def fire_gathers(k, g, sem):
        for h in range(2):
            pltpu.async_copy(
                noun_hbm.at[widx_v.at[pl.ds(k * _TOK + h * _HALF, _HALF)]],
                g.at[pl.ds(h * _HALF, _HALF)], sem)

    def drain_gathers(g, sem):
        for h in range(2):
            pltpu.make_async_copy(
                noun_hbm.at[pl.ds(0, _HALF)],
                g.at[pl.ds(h * _HALF, _HALF)], sem).wait()

    def fire_writes(k, o, sem):
        row0 = (wid * _B_PER_W + k * _NB) * MAX_LEN
        pltpu.async_copy(o, out_hbm.at[pl.ds(row0, _OROWS)], sem)

    def drain_writes(o, sem):
        pltpu.make_async_copy(o, out_hbm.at[pl.ds(0, _OROWS)], sem).wait()

    def add_assemble(k, g, o, dv):
        def batch(j, dvc):
            koff = k * _TOK + j * L_TOK
            ca = (cls_v[pl.ds(koff, 16)] & 1) > 0
            cb = (cls_v[pl.ds(koff + 8, 16)] & 1) > 0
            for l in range(L_TOK):
                half, lane = (ca, l) if l < 16 else (cb, l - 8)
                oh = lax.broadcasted_iota(jnp.int32, (16,), 0) == lane
                cnt = plsc.all_reduce_population_count(half & oh)
                cvf = cnt.astype(jnp.float32)
                tr = j * L_TOK + l
                orow = j * MAX_LEN + 1 + l
                for q in range(D // 16):
                    sl = pl.ds(q * 16, 16)
                    o[orow, sl] = g[tr, sl] + (base_v[l, sl] + cvf * dvc[q])
            return dvc

        return lax.fori_loop(0, _NB, batch, dv)

    def process(k, g, o, semg, semw, gn, semgn, fire_next, drain_w, dv):
        pl.when(fire_next)(lambda: fire_gathers(k + 1, gn, semgn))
        drain_gathers(g, semg)
        pl.when(drain_w)(lambda: drain_writes(o, semw))
        dv = add_assemble(k, g, o, dv)
        fire_writes(k, o, semw)
        return dv

    pltpu.sync_copy(words_hbm.at[pl.ds(wtok, _WTOK)], widx_v)
    pltpu.sync_copy(cls_hbm.at[pl.ds(wtok, _WTOK)], cls_v)
    pltpu.sync_copy(base_hbm, base_v)
    pltpu.sync_copy(dvec_hbm, dvec_v)
    pltpu.sync_copy(spec_hbm, spec_v)
    for ov in (o0, o1):
        for j in range(_NB):
            for q in range(D // 16):
                sl = pl.ds(q * 16, 16)
                ov[j * MAX_LEN, sl] = spec_v[0, sl]
    dv = tuple(dvec_v[0, pl.ds(q * 16, 16)] for q in range(D // 16))
    fire_gathers(0, g0, sem_g0)

    true_ = jnp.bool_(True)

    def pair(i, dvc):
        k = 2 * i
        late = i >= 1
        dvc = process(k, g0, o0, sem_g0, sem_w0, g1, sem_g1, true_, late, dvc)
        dvc = process(k + 1, g1, o1, sem_g1, sem_w1, g0, sem_g0,
                      i < _CHUNKS // 2 - 1, late, dvc)
        return dvc

    lax.fori_loop(0, _CHUNKS // 2, pair, dv)
    drain_writes(o0, sem_w0)
    drain_writes(o1, sem_w1)


def kernel(words, classes, noun_table, class_table, special_emb):
    words_flat = words.astype(jnp.int32).reshape(-1)
    cls_flat = classes.astype(jnp.int32).reshape(-1)
    pe = jnp.asarray(_PE[:L_TOK])
    base = pe + class_table[0][None, :]
    dvec = (class_table[1] - class_table[0])[None, :]

    mesh = plsc.VectorSubcoreMesh(core_axis_name="c", subcore_axis_name="s")
    run = functools.partial(
        pl.kernel,
        mesh=mesh,
        compiler_params=pltpu.CompilerParams(
            use_tc_tiling_on_sc=False, needs_layout_passes=False),
        out_type=jax.ShapeDtypeStruct((B * MAX_LEN, D), jnp.float32),
        scratch_types=[
            pltpu.VMEM((_WTOK,), jnp.int32),
            pltpu.VMEM((_WTOK,), jnp.int32),
            pltpu.VMEM((L_TOK, D), jnp.float32),
            pltpu.VMEM((1, D), jnp.float32),
            pltpu.VMEM((1, D), jnp.float32),
            pltpu.VMEM((_TOK, D), jnp.float32),
            pltpu.VMEM((_TOK, D), jnp.float32),
            pltpu.VMEM((_OROWS, D), jnp.float32),
            pltpu.VMEM((_OROWS, D), jnp.float32),
            pltpu.SemaphoreType.DMA,
            pltpu.SemaphoreType.DMA,
            pltpu.SemaphoreType.DMA,
            pltpu.SemaphoreType.DMA,
        ],
    )(_sc_body)
    out = run(words_flat, cls_flat, noun_table, base, dvec, special_emb)
    return out.reshape(B, MAX_LEN, D)


# speedup vs baseline: 3.6207x; 1.2401x over previous
import functools
import math

import jax
import jax.numpy as jnp
import numpy as np
from jax import lax
from jax.experimental import pallas as pl
from jax.experimental.pallas import tpu as pltpu
from jax.experimental.pallas import tpu_sc as plsc

VOCAB = 100000
D = 128
L_TOK = 24
B = 4096
MAX_LEN = 25


def _pe_const(max_len, d_model):
    position = np.arange(0, max_len, dtype=np.float32)[:, None]
    div_term = np.exp(
        np.arange(0, d_model, 2).astype(np.float32) * (-math.log(10000.0) / d_model)
    )
    pe = np.zeros((max_len, d_model), dtype=np.float32)
    pe[:, 0::2] = np.sin(position * div_term)
    pe[:, 1::2] = np.cos(position * div_term)
    return pe


_PE = _pe_const(MAX_LEN, D)

_INFO = plsc.get_sparse_core_info()
_NC = _INFO.num_cores
_NS = _INFO.num_subcores
_NW = _NC * _NS

_B_PER_W = B // _NW
_NB = 8
_CHUNKS = _B_PER_W // _NB
_TOK = _NB * L_TOK
_WTOK = _B_PER_W * L_TOK
_HALF = _TOK // 2
_OROWS = _NB * MAX_LEN

_ONEHOT = [np.eye(16, dtype=np.bool_)[i] for i in range(16)]


def _sc_body(words_hbm, cls_hbm, noun_hbm, base_hbm, dvec_hbm, spec_hbm,
             out_hbm,
             widx_v, cls_v, base_v, dvec_v, spec_v,
             g0, g1, o0, o1,
             sem_g0, sem_g1, sem_w0, sem_w1):
    wid = lax.axis_index("s") * _NC + lax.axis_index("c")
    wtok = wid * _WTOK

    def fire_gathers(k, g, sem):
        for h in range(2):
            pltpu.async_copy(
                noun_hbm.at[widx_v.at[pl.ds(k * _TOK + h * _HALF, _HALF)]],
                g.at[pl.ds(h * _HALF, _HALF)], sem)

    def drain_gathers(g, sem):
        for h in range(2):
            pltpu.make_async_copy(
                noun_hbm.at[pl.ds(0, _HALF)],
                g.at[pl.ds(h * _HALF, _HALF)], sem).wait()

    def fire_writes(k, o, sem):
        b0 = wid * _B_PER_W + k * _NB
        for j in range(_NB):
            pltpu.async_copy(o.at[pl.ds(j * MAX_LEN, MAX_LEN)],
                             out_hbm.at[b0 + j], sem)

    def drain_writes(o, sem):
        for j in range(_NB):
            pltpu.make_async_copy(o.at[pl.ds(j * MAX_LEN, MAX_LEN)],
                                  out_hbm.at[0], sem).wait()

    def add_assemble(k, g, o, dv):
        def batch(j, dvc):
            koff = k * _TOK + j * L_TOK
            ca = (cls_v[pl.ds(koff, 16)] & 1) > 0
            cb = (cls_v[pl.ds(koff + 8, 16)] & 1) > 0
            for l in range(L_TOK):
                half, lane = (ca, l) if l < 16 else (cb, l - 8)
                oh = lax.broadcasted_iota(jnp.int32, (16,), 0) == lane
                cnt = plsc.all_reduce_population_count(half & oh)
                cvf = cnt.astype(jnp.float32)
                tr = j * L_TOK + l
                orow = j * MAX_LEN + 1 + l
                for q in range(D // 16):
                    sl = pl.ds(q * 16, 16)
                    o[orow, sl] = g[tr, sl] + (base_v[l, sl] + cvf * dvc[q])
            return dvc

        return lax.fori_loop(0, _NB, batch, dv)

    def process(k, g, o, semg, semw, gn, semgn, fire_next, drain_w, dv):
        pl.when(fire_next)(lambda: fire_gathers(k + 1, gn, semgn))
        drain_gathers(g, semg)
        pl.when(drain_w)(lambda: drain_writes(o, semw))
        dv = add_assemble(k, g, o, dv)
        fire_writes(k, o, semw)
        return dv

    pltpu.sync_copy(words_hbm.at[pl.ds(wtok, _WTOK)], widx_v)
    pltpu.sync_copy(cls_hbm.at[pl.ds(wtok, _WTOK)], cls_v)
    pltpu.sync_copy(base_hbm, base_v)
    pltpu.sync_copy(dvec_hbm, dvec_v)
    pltpu.sync_copy(spec_hbm, spec_v)
    for ov in (o0, o1):
        for j in range(_NB):
            for q in range(D // 16):
                sl = pl.ds(q * 16, 16)
                ov[j * MAX_LEN, sl] = spec_v[0, sl]
    dv = tuple(dvec_v[0, pl.ds(q * 16, 16)] for q in range(D // 16))
    fire_gathers(0, g0, sem_g0)

    true_ = jnp.bool_(True)

    def pair(i, dvc):
        k = 2 * i
        late = i >= 1
        dvc = process(k, g0, o0, sem_g0, sem_w0, g1, sem_g1, true_, late, dvc)
        dvc = process(k + 1, g1, o1, sem_g1, sem_w1, g0, sem_g0,
                      i < _CHUNKS // 2 - 1, late, dvc)
        return dvc

    lax.fori_loop(0, _CHUNKS // 2, pair, dv)
    drain_writes(o0, sem_w0)
    drain_writes(o1, sem_w1)


def kernel(words, classes, noun_table, class_table, special_emb):
    words_flat = words.astype(jnp.int32).reshape(-1)
    cls_flat = classes.astype(jnp.int32).reshape(-1)
    pe = jnp.asarray(_PE[:L_TOK])
    base = pe + class_table[0][None, :]
    dvec = (class_table[1] - class_table[0])[None, :]

    mesh = plsc.VectorSubcoreMesh(core_axis_name="c", subcore_axis_name="s")
    run = functools.partial(
        pl.kernel,
        mesh=mesh,
        compiler_params=pltpu.CompilerParams(needs_layout_passes=False),
        out_type=jax.ShapeDtypeStruct((B, MAX_LEN, D), jnp.float32),
        scratch_types=[
            pltpu.VMEM((_WTOK,), jnp.int32),
            pltpu.VMEM((_WTOK,), jnp.int32),
            pltpu.VMEM((L_TOK, D), jnp.float32),
            pltpu.VMEM((1, D), jnp.float32),
            pltpu.VMEM((1, D), jnp.float32),
            pltpu.VMEM((_TOK, D), jnp.float32),
            pltpu.VMEM((_TOK, D), jnp.float32),
            pltpu.VMEM((_OROWS, D), jnp.float32),
            pltpu.VMEM((_OROWS, D), jnp.float32),
            pltpu.SemaphoreType.DMA,
            pltpu.SemaphoreType.DMA,
            pltpu.SemaphoreType.DMA,
            pltpu.SemaphoreType.DMA,
        ],
    )(_sc_body)
    return run(words_flat, cls_flat, noun_table, base, dvec, special_emb)
